# baseline (speedup 1.0000x reference)
import functools

import jax
import jax.numpy as jnp
from jax import lax
from jax.experimental import pallas as pl
from jax.experimental.pallas import tpu as pltpu
from jax.experimental.pallas import tpu_sc as plsc

VOCAB = 1000000
EMBED = 32
NCLS = 2
BATCH = 4096
SEQ = 200

NC = 2
NS = 16
L = 16
NW = NC * NS
BPW = BATCH // NW
CH = 4
NCHUNK = BPW // CH
S2 = 2 * SEQ
NT = S2 // L
SPLITS = ((0, 128), (128, 128), (256, 128), (384, 16))

BKV = 8000
PBLK = 16384
CLS1 = 8192
NBLK = VOCAB // BKV

_mesh = plsc.VectorSubcoreMesh(core_axis_name="c", subcore_axis_name="s")


def _proj_body(t_ref, w_ref, b_ref, p_ref):
    t = t_ref[...]
    w = w_ref[...] * (1.0 / SEQ)
    bb = b_ref[...] * (1.0 / SEQ)
    p_ref[pl.ds(0, BKV)] = jnp.sum(t * w[:, 0], axis=1) + bb[0, 0]
    p_ref[pl.ds(CLS1, BKV)] = jnp.sum(t * w[:, 1], axis=1) + bb[0, 1]


_proj = pl.pallas_call(
    _proj_body,
    grid=(VOCAB // BKV,),
    in_specs=[
        pl.BlockSpec((BKV, EMBED), lambda i: (i, 0)),
        pl.BlockSpec((EMBED, NCLS), lambda i: (0, 0)),
        pl.BlockSpec((1, NCLS), lambda i: (0, 0)),
    ],
    out_specs=pl.BlockSpec((PBLK,), lambda i: (i,)),
    out_shape=jax.ShapeDtypeStruct((NBLK * PBLK,), jnp.float32),
)

_ROT_DNUMS = lax.GatherDimensionNumbers(
    offset_dims=(), collapsed_slice_dims=(0,), start_index_map=(0,))


def _rot_add(v, iota, k):
    perm = (iota + k) % L
    r = lax.gather(v, perm[:, None], _ROT_DNUMS, slice_sizes=(1,),
                   mode=lax.GatherScatterMode.PROMISE_IN_BOUNDS)
    return v + r


@functools.partial(
    pl.kernel,
    mesh=_mesh,
    compiler_params=pltpu.CompilerParams(use_tc_tiling_on_sc=False),
    out_type=jax.ShapeDtypeStruct((BATCH, L), jnp.float32),
    scratch_types=[
        pltpu.VMEM((CH, S2), jnp.int32),
        pltpu.VMEM((CH, S2), jnp.float32),
        pltpu.VMEM((BPW, L), jnp.float32),
        pltpu.SemaphoreType.DMA,
    ],
)
def _pool_gather(idx_hbm, pf_hbm, out_hbm, idx_v, rows_v, acc_v, sem):
    wid = lax.axis_index("s") * NC + lax.axis_index("c")
    base = wid * BPW
    iota = lax.iota(jnp.int32, L)

    def chunk_body(ci, carry):
        row0 = base + ci * CH
        pltpu.sync_copy(idx_hbm.at[pl.ds(row0, CH)], idx_v)
        cps = []
        for r in range(CH):
            for off, n in SPLITS:
                cps.append(pltpu.async_copy(
                    pf_hbm.at[idx_v.at[r, pl.ds(off, n)]],
                    rows_v.at[r, pl.ds(off, n)], sem))
        for cp in cps:
            cp.wait()
        for r in range(CH):
            acc = rows_v[r, pl.ds(0, L)]
            for t in range(1, NT):
                acc = acc + rows_v[r, pl.ds(t * L, L)]
            acc = _rot_add(acc, iota, 2)
            acc = _rot_add(acc, iota, 4)
            acc = _rot_add(acc, iota, 8)
            acc_v[ci * CH + r, :] = acc
        return carry

    lax.fori_loop(0, NCHUNK, chunk_body, 0)
    pltpu.sync_copy(acc_v, out_hbm.at[pl.ds(base, BPW)])


def kernel(x, table, W, b):
    xi = x.astype(jnp.int32)
    flat0 = PBLK * (xi // BKV) + xi % BKV
    idx2 = jnp.stack([flat0, flat0 + CLS1], axis=-1).reshape(BATCH, S2)
    p = _proj(table, W, b.reshape(1, NCLS))
    out16 = _pool_gather(idx2, p)
    return out16[:, :NCLS]


# speedup vs baseline: 2.0502x; 2.0502x over previous
import functools

import jax
import jax.numpy as jnp
from jax import lax
from jax.experimental import pallas as pl
from jax.experimental.pallas import tpu as pltpu
from jax.experimental.pallas import tpu_sc as plsc

VOCAB = 1000000
EMBED = 32
NCLS = 2
BATCH = 4096
SEQ = 200

NC = 2
NS = 16
L = 16
NW = NC * NS
BPW = BATCH // NW
CH = 4
NCHUNK = BPW // CH
S2 = 2 * SEQ
SPLITS = ((0, 128), (128, 128), (256, 128), (384, 16))
UNROLL = 4

_mesh = plsc.VectorSubcoreMesh(core_axis_name="c", subcore_axis_name="s")


@functools.partial(
    pl.kernel,
    mesh=_mesh,
    compiler_params=pltpu.CompilerParams(use_tc_tiling_on_sc=False),
    out_type=jax.ShapeDtypeStruct((BATCH, EMBED), jnp.float32),
    scratch_types=[
        pltpu.VMEM((CH, S2), jnp.int32),
        pltpu.VMEM((CH, S2, L), jnp.float32),
        pltpu.VMEM((BPW, EMBED), jnp.float32),
        pltpu.SemaphoreType.DMA,
    ],
)
def _pooled_sum(idx_hbm, t16_hbm, out_hbm, idx_v, rows_v, acc_v, sem):
    wid = lax.axis_index("s") * NC + lax.axis_index("c")
    base = wid * BPW

    def chunk_body(ci, carry):
        row0 = base + ci * CH
        pltpu.sync_copy(idx_hbm.at[pl.ds(row0, CH)], idx_v)
        cps = []
        for r in range(CH):
            for off, n in SPLITS:
                cps.append(pltpu.async_copy(
                    t16_hbm.at[idx_v.at[r, pl.ds(off, n)]],
                    rows_v.at[r, pl.ds(off, n)], sem))
        for cp in cps:
            cp.wait()
        for r in range(CH):
            def sbody(i, acc, _r=r):
                a0, a1 = acc
                for k in range(UNROLL):
                    s = i * UNROLL + k
                    a0 = a0 + rows_v[_r, 2 * s, :]
                    a1 = a1 + rows_v[_r, 2 * s + 1, :]
                return a0, a1
            z = jnp.zeros((L,), jnp.float32)
            a0, a1 = lax.fori_loop(0, SEQ // UNROLL, sbody, (z, z))
            acc_v[ci * CH + r, pl.ds(0, L)] = a0
            acc_v[ci * CH + r, pl.ds(L, L)] = a1
        return carry

    lax.fori_loop(0, NCHUNK, chunk_body, 0)
    pltpu.sync_copy(acc_v, out_hbm.at[pl.ds(base, BPW)])


def _head_body(p_ref, w_ref, b_ref, o_ref):
    pooled = p_ref[...] * (1.0 / SEQ)
    o_ref[...] = (
        jnp.dot(pooled, w_ref[...], preferred_element_type=jnp.float32,
                precision=lax.Precision.HIGHEST)
        + b_ref[...]
    )


_head = pl.pallas_call(
    _head_body,
    out_shape=jax.ShapeDtypeStruct((BATCH, NCLS), jnp.float32),
)


def kernel(x, table, W, b):
    xi = x.astype(jnp.int32)
    idx2 = jnp.stack([xi * 2, xi * 2 + 1], axis=-1).reshape(BATCH, S2)
    t16 = table.reshape(VOCAB * NCLS, L)
    pooled = _pooled_sum(idx2, t16)
    return _head(pooled, W, b.reshape(1, NCLS))
